# baseline (speedup 1.0000x reference)
import functools

import jax
import jax.numpy as jnp
from jax.experimental import pallas as pl
from jax.experimental.pallas import tpu as pltpu


def _fused_kernel(K, idx_ref, x_ref, w_ref, out_ref, xg_ref, xg_bf_ref):
    b = pl.program_id(0)
    e = pl.program_id(1)
    jb = pl.program_id(2)

    @pl.when(jb == 0)
    def _gather():
        def body(k, carry):
            t = idx_ref[b, e, k]
            xg_ref[pl.ds(k, 1), :] = x_ref[0, pl.ds(t, 1), :]
            return carry

        jax.lax.fori_loop(0, K, body, 0, unroll=8)
        xg_bf_ref[...] = xg_ref[...].astype(jnp.bfloat16)

    out_ref[0, 0] = jnp.dot(
        xg_bf_ref[...],
        w_ref[0].astype(jnp.bfloat16),
        preferred_element_type=jnp.float32,
    )


@functools.partial(jax.jit, static_argnames=("jblk", "interpret"))
def _run(x, indices, W, jblk=512, interpret=False):
    B, T, I = x.shape
    _, E, K = indices.shape
    J = W.shape[2]
    grid = (B, E, J // jblk)
    grid_spec = pltpu.PrefetchScalarGridSpec(
        num_scalar_prefetch=1,
        grid=grid,
        in_specs=[
            pl.BlockSpec((1, T, I), lambda b, e, jb, idx: (b, 0, 0)),
            pl.BlockSpec((1, I, jblk), lambda b, e, jb, idx: (e, 0, jb)),
        ],
        out_specs=pl.BlockSpec((1, 1, K, jblk), lambda b, e, jb, idx: (b, e, 0, jb)),
        scratch_shapes=[
            pltpu.VMEM((K, I), jnp.float32),
            pltpu.VMEM((K, I), jnp.bfloat16),
        ],
    )
    fn = pl.pallas_call(
        functools.partial(_fused_kernel, K),
        grid_spec=grid_spec,
        out_shape=jax.ShapeDtypeStruct((B, E, K, J), jnp.float32),
        compiler_params=pltpu.CompilerParams(
            dimension_semantics=("arbitrary", "arbitrary", "arbitrary"),
        ),
        interpret=interpret,
    )
    return fn(indices, x, W)


def kernel(x, indices, W):
    return _run(x, indices, W)


# speedup vs baseline: 1.2223x; 1.2223x over previous
import functools

import jax
import jax.numpy as jnp
from jax.experimental import pallas as pl
from jax.experimental.pallas import tpu as pltpu


def _fused_kernel(E, K, B, JB, idx_ref, x_ref, w_ref, out_ref,
                  land_ref, xg_ref, wbf_ref, sems):
    e = pl.program_id(0)
    jb = pl.program_id(1)
    b = pl.program_id(2)

    def enqueue(eg, bg):
        def body(k, carry):
            t = idx_ref[bg, eg, k]
            pltpu.make_async_copy(
                x_ref.at[bg, pl.ds(t, 1), :],
                land_ref.at[bg, pl.ds(k, 1), :],
                sems.at[bg],
            ).start()
            return carry

        jax.lax.fori_loop(0, K, body, 0, unroll=8)

    @pl.when((e == 0) & (jb == 0) & (b == 0))
    def _prologue():
        for bg in range(B):
            enqueue(0, bg)

    @pl.when(jb == 0)
    def _land():
        pltpu.make_async_copy(
            x_ref.at[0, pl.ds(0, K), :],
            land_ref.at[b],
            sems.at[b],
        ).wait()
        xg_ref[b] = land_ref[b].astype(jnp.bfloat16)

    @pl.when(b == 0)
    def _wcast():
        wbf_ref[...] = w_ref[0].astype(jnp.bfloat16)

    @pl.when((jb == JB - 1) & (e < E - 1))
    def _prefetch_next():
        enqueue(e + 1, b)

    out_ref[0, 0] = jnp.dot(
        xg_ref[b], wbf_ref[...], preferred_element_type=jnp.float32
    )


@functools.partial(jax.jit, static_argnames=("jblk", "interpret"))
def _run(x, indices, W, jblk=1024, interpret=False):
    B, T, I = x.shape
    _, E, K = indices.shape
    J = W.shape[2]
    JB = J // jblk
    grid = (E, JB, B)
    grid_spec = pltpu.PrefetchScalarGridSpec(
        num_scalar_prefetch=1,
        grid=grid,
        in_specs=[
            pl.BlockSpec(memory_space=pl.ANY),
            pl.BlockSpec((1, I, jblk), lambda e, jb, b, idx: (e, 0, jb)),
        ],
        out_specs=pl.BlockSpec(
            (1, 1, K, jblk), lambda e, jb, b, idx: (b, e, 0, jb)
        ),
        scratch_shapes=[
            pltpu.VMEM((B, K, I), jnp.float32),
            pltpu.VMEM((B, K, I), jnp.bfloat16),
            pltpu.VMEM((I, jblk), jnp.bfloat16),
            pltpu.SemaphoreType.DMA((B,)),
        ],
    )
    fn = pl.pallas_call(
        functools.partial(_fused_kernel, E, K, B, JB),
        grid_spec=grid_spec,
        out_shape=jax.ShapeDtypeStruct((B, E, K, J), jnp.float32),
        compiler_params=pltpu.CompilerParams(
            dimension_semantics=("arbitrary", "arbitrary", "arbitrary"),
        ),
        interpret=interpret,
    )
    return fn(indices, x, W)


def kernel(x, indices, W):
    return _run(x, indices, W)


# speedup vs baseline: 1.2647x; 1.0347x over previous
import functools

import jax
import jax.numpy as jnp
from jax.experimental import pallas as pl
from jax.experimental.pallas import tpu as pltpu


def _fused_kernel(E, K, B, JB, idx_ref, x_ref, w_ref, out_ref,
                  land_ref, sems):
    e = pl.program_id(0)
    jb = pl.program_id(1)
    b = pl.program_id(2)

    def enqueue(eg, bg):
        def body(k, carry):
            t = idx_ref[bg, eg, k]
            pltpu.make_async_copy(
                x_ref.at[bg, pl.ds(t, 1), :],
                land_ref.at[bg, pl.ds(k, 1), :],
                sems.at[bg],
            ).start()
            return carry

        jax.lax.fori_loop(0, K, body, 0, unroll=8)

    @pl.when((e == 0) & (jb == 0) & (b == 0))
    def _prologue():
        for bg in range(B):
            enqueue(0, bg)

    @pl.when(jb == 0)
    def _land():
        pltpu.make_async_copy(
            x_ref.at[0, pl.ds(0, K), :],
            land_ref.at[b],
            sems.at[b],
        ).wait()

    out_ref[0, 0] = jnp.dot(
        land_ref[b], w_ref[0], preferred_element_type=jnp.float32
    )

    @pl.when((jb == JB - 1) & (e < E - 1))
    def _prefetch_next():
        enqueue(e + 1, b)


@functools.partial(jax.jit, static_argnames=("jblk", "interpret"))
def _run(x, indices, W, jblk=1024, interpret=False):
    B, T, I = x.shape
    _, E, K = indices.shape
    J = W.shape[2]
    JB = J // jblk
    grid = (E, JB, B)
    grid_spec = pltpu.PrefetchScalarGridSpec(
        num_scalar_prefetch=1,
        grid=grid,
        in_specs=[
            pl.BlockSpec(memory_space=pl.ANY),
            pl.BlockSpec((1, I, jblk), lambda e, jb, b, idx: (e, 0, jb)),
        ],
        out_specs=pl.BlockSpec(
            (1, 1, K, jblk), lambda e, jb, b, idx: (b, e, 0, jb)
        ),
        scratch_shapes=[
            pltpu.VMEM((B, K, I), jnp.float32),
            pltpu.SemaphoreType.DMA((B,)),
        ],
    )
    fn = pl.pallas_call(
        functools.partial(_fused_kernel, E, K, B, JB),
        grid_spec=grid_spec,
        out_shape=jax.ShapeDtypeStruct((B, E, K, J), jnp.float32),
        compiler_params=pltpu.CompilerParams(
            dimension_semantics=("arbitrary", "arbitrary", "arbitrary"),
        ),
        interpret=interpret,
    )
    return fn(indices, x, W)


def kernel(x, indices, W):
    return _run(x, indices, W)


# speedup vs baseline: 1.3761x; 1.0881x over previous
import functools

import jax
import jax.numpy as jnp
from jax.experimental import pallas as pl
from jax.experimental.pallas import tpu as pltpu


def _fused_kernel(E, K, B, JB, idx_ref, x_ref, w_ref, out_ref,
                  land_ref, sems):
    e = pl.program_id(0)
    jb = pl.program_id(1)
    b = pl.program_id(2)

    def enqueue(eg, bg):
        def body(k, carry):
            t = idx_ref[bg, eg, k]
            pltpu.make_async_copy(
                x_ref.at[bg, pl.ds(t, 1), :],
                land_ref.at[bg, pl.ds(k, 1), :],
                sems.at[bg],
            ).start()
            return carry

        jax.lax.fori_loop(0, K, body, 0, unroll=8)

    @pl.when((e == 0) & (jb == 0) & (b == 0))
    def _prologue():
        for bg in range(B):
            enqueue(0, bg)

    @pl.when(jb == 0)
    def _land():
        pltpu.make_async_copy(
            x_ref.at[0, pl.ds(0, K), :],
            land_ref.at[b],
            sems.at[b],
        ).wait()

    out_ref[0, 0] = jnp.dot(
        land_ref[b], w_ref[0], preferred_element_type=jnp.float32
    )

    @pl.when((jb == JB - 1) & (e < E - 1))
    def _prefetch_next():
        enqueue(e + 1, b)


@functools.partial(jax.jit, static_argnames=("jblk", "interpret"))
def _run(x, indices, W, jblk=2048, interpret=False):
    B, T, I = x.shape
    _, E, K = indices.shape
    J = W.shape[2]
    JB = J // jblk
    grid = (E, JB, B)
    grid_spec = pltpu.PrefetchScalarGridSpec(
        num_scalar_prefetch=1,
        grid=grid,
        in_specs=[
            pl.BlockSpec(memory_space=pl.ANY),
            pl.BlockSpec((1, I, jblk), lambda e, jb, b, idx: (e, 0, jb)),
        ],
        out_specs=pl.BlockSpec(
            (1, 1, K, jblk), lambda e, jb, b, idx: (b, e, 0, jb)
        ),
        scratch_shapes=[
            pltpu.VMEM((B, K, I), jnp.float32),
            pltpu.SemaphoreType.DMA((B,)),
        ],
    )
    fn = pl.pallas_call(
        functools.partial(_fused_kernel, E, K, B, JB),
        grid_spec=grid_spec,
        out_shape=jax.ShapeDtypeStruct((B, E, K, J), jnp.float32),
        compiler_params=pltpu.CompilerParams(
            dimension_semantics=("arbitrary", "arbitrary", "arbitrary"),
        ),
        interpret=interpret,
    )
    return fn(indices, x, W)


def kernel(x, indices, W):
    return _run(x, indices, W)


# speedup vs baseline: 1.4241x; 1.0349x over previous
import functools

import jax
import jax.numpy as jnp
from jax.experimental import pallas as pl
from jax.experimental.pallas import tpu as pltpu


def _fused_kernel(E, K, B, JB, idx_ref, x_ref, w_ref, out_ref,
                  land_ref, sems):
    e = pl.program_id(0)
    jb = pl.program_id(1)
    b = pl.program_id(2)

    def enqueue(eg, bg):
        def body(k, carry):
            t = idx_ref[bg, eg, k]
            pltpu.make_async_copy(
                x_ref.at[pl.ds(t, 1), :],
                land_ref.at[bg, pl.ds(k, 1), :],
                sems.at[bg],
            ).start()
            return carry

        jax.lax.fori_loop(0, K, body, 0, unroll=16)

    @pl.when((e == 0) & (jb == 0) & (b == 0))
    def _prologue():
        for bg in range(B):
            enqueue(0, bg)

    @pl.when(jb == 0)
    def _land():
        pltpu.make_async_copy(
            x_ref.at[pl.ds(0, K), :],
            land_ref.at[b],
            sems.at[b],
        ).wait()

    out_ref[0, 0] = jnp.dot(
        land_ref[b], w_ref[0], preferred_element_type=jnp.float32
    )

    @pl.when((jb == JB - 1) & (e < E - 1))
    def _prefetch_next():
        enqueue(e + 1, b)


@functools.partial(jax.jit, static_argnames=("jblk", "interpret"))
def _run(x, indices, W, jblk=2048, interpret=False):
    B, T, I = x.shape
    _, E, K = indices.shape
    J = W.shape[2]
    JB = J // jblk
    grid = (E, JB, B)
    x2 = x.reshape(B * T, I)
    idx2 = indices + (jnp.arange(B, dtype=jnp.int32) * T)[:, None, None]
    grid_spec = pltpu.PrefetchScalarGridSpec(
        num_scalar_prefetch=1,
        grid=grid,
        in_specs=[
            pl.BlockSpec(memory_space=pl.ANY),
            pl.BlockSpec((1, I, jblk), lambda e, jb, b, idx: (e, 0, jb)),
        ],
        out_specs=pl.BlockSpec(
            (1, 1, K, jblk), lambda e, jb, b, idx: (b, e, 0, jb)
        ),
        scratch_shapes=[
            pltpu.VMEM((B, K, I), jnp.float32),
            pltpu.SemaphoreType.DMA((B,)),
        ],
    )
    fn = pl.pallas_call(
        functools.partial(_fused_kernel, E, K, B, JB),
        grid_spec=grid_spec,
        out_shape=jax.ShapeDtypeStruct((B, E, K, J), jnp.float32),
        compiler_params=pltpu.CompilerParams(
            dimension_semantics=("arbitrary", "arbitrary", "arbitrary"),
        ),
        interpret=interpret,
    )
    return fn(idx2, x2, W)


def kernel(x, indices, W):
    return _run(x, indices, W)
